# baseline (speedup 1.0000x reference)
import functools

import jax
import jax.numpy as jnp
from jax import lax
from jax.experimental import pallas as pl
from jax.experimental.pallas import tpu as pltpu
from jax.experimental.pallas import tpu_sc as plsc

_GAMMA = 12.0
_LANES = 16

_SIN_C = (
    0.9999999562127889,
    -0.16666631913872146,
    0.00833289061409179,
    -0.00019820756363012935,
    2.712799827662477e-06,
    -2.0872664575493573e-08,
)
_COS_C = (
    0.9999999922898474,
    -0.49999991770959235,
    0.04166652433757078,
    -0.0013887970265659048,
    2.4773420813397368e-05,
    -2.711333772339074e-07,
    1.7368996050969864e-09,
)


def _sincos(x):
    u = x * x
    s = jnp.float32(_SIN_C[5])
    for c in _SIN_C[4::-1]:
        s = s * u + jnp.float32(c)
    s = s * x
    c = jnp.float32(_COS_C[6])
    for cc in _COS_C[5::-1]:
        c = c * u + jnp.float32(cc)
    return s, c


def _sqrt(x):
    i = lax.bitcast_convert_type(x, jnp.int32)
    i = jnp.int32(0x5F3759DF) - lax.shift_right_arithmetic(i, 1)
    y = lax.bitcast_convert_type(i, jnp.float32)
    for _ in range(3):
        t = jnp.float32(0.5) * x * y
        y = y * (jnp.float32(1.5) - t * y)
    return x * y


def _make_kernel(batch, dim, n_workers, bpw, n_chunks, chunk):
    mesh = plsc.VectorSubcoreMesh(core_axis_name="c", subcore_axis_name="s")
    nc = mesh.num_cores

    @functools.partial(
        pl.kernel,
        out_type=jax.ShapeDtypeStruct((batch,), jnp.float32),
        mesh=mesh,
        compiler_params=pltpu.CompilerParams(
            needs_layout_passes=False, use_tc_tiling_on_sc=False),
        scratch_types=[
            pltpu.VMEM((n_chunks, chunk), jnp.int32),
            pltpu.VMEM((n_chunks, chunk), jnp.int32),
            pltpu.VMEM((n_chunks, chunk), jnp.int32),
            pltpu.VMEM((bpw, dim), jnp.float32),
            pltpu.VMEM((bpw, dim), jnp.float32),
            pltpu.VMEM((bpw, dim), jnp.float32),
            pltpu.VMEM((bpw, dim), jnp.float32),
            pltpu.VMEM((bpw, dim), jnp.float32),
            pltpu.VMEM((_LANES * bpw,), jnp.float32),
            pltpu.VMEM((bpw,), jnp.float32),
            pltpu.SemaphoreType.DMA,
        ],
    )
    def rotate_kernel(h_idx, r_idx, t_idx, ent, ent_im, rel, out,
                      ih, ir, it, bh_re, bh_im, br, bt_re, bt_im, tr, sc, sem):
        wid = lax.axis_index("s") * nc + lax.axis_index("c")
        base = wid * bpw

        pltpu.sync_copy(h_idx.at[wid], ih)
        pltpu.sync_copy(r_idx.at[wid], ir)
        pltpu.sync_copy(t_idx.at[wid], it)

        copies = []
        for j in range(n_chunks):
            d = pl.ds(j * chunk, chunk)
            copies.append(pltpu.async_copy(ent.at[ih.at[j]], bh_re.at[d], sem))
            copies.append(pltpu.async_copy(ent_im.at[ih.at[j]], bh_im.at[d], sem))
            copies.append(pltpu.async_copy(rel.at[ir.at[j]], br.at[d], sem))
            copies.append(pltpu.async_copy(ent.at[it.at[j]], bt_re.at[d], sem))
            copies.append(pltpu.async_copy(ent_im.at[it.at[j]], bt_im.at[d], sem))
        for cp in copies:
            cp.wait()

        iota = lax.iota(jnp.int32, _LANES)

        def row_body(b, carry):
            acc = None
            for half in range(dim // _LANES):
                d = pl.ds(half * _LANES, _LANES)
                sn, cs = _sincos(br[b, d])
                hre = bh_re[b, d]
                him = bh_im[b, d]
                s_re = hre * cs - him * sn - bt_re[b, d]
                s_im = hre * sn + him * cs - bt_im[b, d]
                m = _sqrt(s_re * s_re + s_im * s_im)
                acc = m if acc is None else acc + m
            plsc.store_scatter(tr, [iota * bpw + b], acc)
            return carry

        lax.fori_loop(0, bpw, row_body, 0)

        def grp_body(g, carry):
            acc = tr[pl.ds(g * _LANES, _LANES)]
            for k in range(1, _LANES):
                acc = acc + tr[pl.ds(k * bpw + g * _LANES, _LANES)]
            sc[pl.ds(g * _LANES, _LANES)] = acc - jnp.float32(_GAMMA)
            return carry

        lax.fori_loop(0, bpw // _LANES, grp_body, 0)

        pltpu.sync_copy(sc, out.at[pl.ds(base, bpw)])

    return rotate_kernel


def kernel(pos_sample, ent_embd, ent_embd_im, rel_embd):
    batch = pos_sample.shape[0]
    dim = ent_embd.shape[1]
    n_workers = 32
    bpw = batch // n_workers
    chunk = 128
    n_chunks = bpw // chunk

    h_idx = pos_sample[:, 0].reshape(n_workers, n_chunks, chunk)
    r_idx = pos_sample[:, 1].reshape(n_workers, n_chunks, chunk)
    t_idx = pos_sample[:, 2].reshape(n_workers, n_chunks, chunk)

    k = _make_kernel(batch, dim, n_workers, bpw, n_chunks, chunk)
    score = k(h_idx, r_idx, t_idx, ent_embd, ent_embd_im, rel_embd)
    return score.reshape(batch, 1)
